# Initial kernel scaffold; baseline (speedup 1.0000x reference)
import jax
import jax.numpy as jnp
from jax.experimental import pallas as pl


def kernel(x, edge_index, W1, a_src1, a_dst1, b1, W2, a_src2, a_dst2, b2, W3, a_src3, a_dst3, b3, Wc, bc):
    raise NotImplementedError("write your pallas kernel here")


# speedup vs baseline: 52.7255x; 52.7255x over previous
import functools

import jax
import jax.numpy as jnp
from jax import lax
from jax.experimental import pallas as pl
from jax.experimental.pallas import tpu as pltpu
from jax.experimental.pallas import tpu_sc as plsc

N = 10000
E = 320000
IN = 128
HID = 16

NC = 2
NS = 16
NW = NC * NS
EPT = E // NW
EB = 80
NBLK = EPT // EB
RPT = 624

NB = 400
NG = N // NB


def _elu(x):
    return jnp.where(x > 0, x, jnp.exp(x) - 1.0)


def _pack_tables(h, a_src, a_dst, H, tab_ref, dtab_ref, cm_ref, k):
    nb = h.shape[0]
    h3 = h.reshape(nb, H, HID)
    As = jnp.sum(h3 * a_src[None, :, :], axis=-1)
    Ad = jnp.sum(h3 * a_dst[None, :, :], axis=-1)
    pad = jnp.zeros((nb, 16 - H), dtype=jnp.float32)
    tab_ref[...] = jnp.concatenate([h, As, pad], axis=1)
    dtab_ref[...] = jnp.concatenate([Ad, pad], axis=1)
    if H == 8:
        parts = [jnp.max(As, axis=0), jnp.max(Ad, axis=0)]
    else:
        z8 = jnp.zeros((8 - H,), dtype=jnp.float32)
        parts = [jnp.max(As, axis=0), z8, jnp.max(Ad, axis=0), z8]
    bm = jnp.concatenate(parts).reshape(1, 16)

    @pl.when(k == 0)
    def _():
        cm_ref[...] = bm

    @pl.when(k > 0)
    def _():
        cm_ref[...] = jnp.maximum(cm_ref[...], bm)


def _pre_first_body(x_ref, W_ref, as_ref, ad_ref, tab_ref, dtab_ref, cm_ref):
    k = pl.program_id(0)
    h = jnp.dot(x_ref[...], W_ref[...], preferred_element_type=jnp.float32)
    _pack_tables(h, as_ref[...], ad_ref[...], 8, tab_ref, dtab_ref, cm_ref, k)


def _pre_mid_body(H, acc_ref, b_ref, W_ref, as_ref, ad_ref,
                  tab_ref, dtab_ref, cm_ref):
    k = pl.program_id(0)
    a = acc_ref[0] + acc_ref[1]
    den = a[:, IN:IN + 8]
    den16 = jnp.broadcast_to(den[:, :, None], (NB, 8, 16)).reshape(NB, IN)
    x = _elu(a[:, :IN] / (den16 + 1e-16) + b_ref[...])
    h = jnp.dot(x, W_ref[...], preferred_element_type=jnp.float32)
    _pack_tables(h, as_ref[...], ad_ref[...], H, tab_ref, dtab_ref, cm_ref, k)


def _fin_body(acc_ref, b_ref, wc_ref, bc_ref, out_ref):
    a = acc_ref[0] + acc_ref[1]
    den = jnp.broadcast_to(a[:, HID:HID + 1], (NB, HID))
    x = _elu(a[:, :HID] / (den + 1e-16) + b_ref[...])
    logits = jnp.sum(x * wc_ref[...], axis=1, keepdims=True) + bc_ref[...]
    out_ref[...] = jax.nn.sigmoid(logits)


def _tc_pre_first(x, W, a_src, a_dst):
    return pl.pallas_call(
        _pre_first_body,
        grid=(NG,),
        in_specs=[
            pl.BlockSpec((NB, IN), lambda k: (k, 0)),
            pl.BlockSpec((IN, IN), lambda k: (0, 0)),
            pl.BlockSpec((8, HID), lambda k: (0, 0)),
            pl.BlockSpec((8, HID), lambda k: (0, 0)),
        ],
        out_specs=[
            pl.BlockSpec((NB, IN + 16), lambda k: (k, 0)),
            pl.BlockSpec((NB, 16), lambda k: (k, 0)),
            pl.BlockSpec((1, 16), lambda k: (0, 0)),
        ],
        out_shape=[
            jax.ShapeDtypeStruct((N, IN + 16), jnp.float32),
            jax.ShapeDtypeStruct((N, 16), jnp.float32),
            jax.ShapeDtypeStruct((1, 16), jnp.float32),
        ],
    )(x, W, a_src, a_dst)


def _tc_pre_mid(acc, b_prev, W, a_src, a_dst, H):
    HC = H * HID
    TW = HC + 16
    return pl.pallas_call(
        functools.partial(_pre_mid_body, H),
        grid=(NG,),
        in_specs=[
            pl.BlockSpec((2, NB, IN + 16), lambda k: (0, k, 0)),
            pl.BlockSpec((1, IN), lambda k: (0, 0)),
            pl.BlockSpec((IN, HC), lambda k: (0, 0)),
            pl.BlockSpec((H, HID), lambda k: (0, 0)),
            pl.BlockSpec((H, HID), lambda k: (0, 0)),
        ],
        out_specs=[
            pl.BlockSpec((NB, TW), lambda k: (k, 0)),
            pl.BlockSpec((NB, 16), lambda k: (k, 0)),
            pl.BlockSpec((1, 16), lambda k: (0, 0)),
        ],
        out_shape=[
            jax.ShapeDtypeStruct((N, TW), jnp.float32),
            jax.ShapeDtypeStruct((N, 16), jnp.float32),
            jax.ShapeDtypeStruct((1, 16), jnp.float32),
        ],
    )(acc, b_prev, W, a_src, a_dst)


def _tc_fin(acc, b3, wcT, bc):
    return pl.pallas_call(
        _fin_body,
        grid=(NG,),
        in_specs=[
            pl.BlockSpec((2, NB, HID + 16), lambda k: (0, k, 0)),
            pl.BlockSpec((1, HID), lambda k: (0, 0)),
            pl.BlockSpec((1, HID), lambda k: (0, 0)),
            pl.BlockSpec((1, 1), lambda k: (0, 0)),
        ],
        out_specs=pl.BlockSpec((NB, 1), lambda k: (k, 0)),
        out_shape=jax.ShapeDtypeStruct((N, 1), jnp.float32),
    )(acc, b3, wcT, bc)


def _sc_body(H, tab_hbm, dtab_hbm, esrc_hbm, edst_hbm, c16_hbm, z_hbm,
             out_hbm, rows, arows, sidx, didx, c16v, acc, gsem, asem):
    HC = H * HID
    cid = lax.axis_index("c")
    sid = lax.axis_index("s")
    wid = sid * NC + cid

    @pl.when(sid < NS - 1)
    def _():
        pltpu.sync_copy(z_hbm.at[pl.ds(0, RPT)], acc.at[pl.ds(sid * RPT, RPT)])

    @pl.when(sid == NS - 1)
    def _():
        pltpu.sync_copy(z_hbm, acc.at[pl.ds((NS - 1) * RPT, N - (NS - 1) * RPT)])

    pltpu.sync_copy(c16_hbm, c16v)
    plsc.subcore_barrier()

    cvec = c16v[...]
    base = wid * EPT

    def block(b, carry):
        off = base + b * EB
        pltpu.sync_copy(esrc_hbm.at[pl.ds(off, EB)], sidx)
        pltpu.sync_copy(edst_hbm.at[pl.ds(off, EB)], didx)
        gcp = pltpu.async_copy(tab_hbm.at[sidx], rows, gsem)
        acp = pltpu.async_copy(dtab_hbm.at[didx], arows, asem)
        gcp.wait()
        acp.wait()

        def edge(e, c2):
            s16 = rows[e, pl.ds(HC, 16)] + arows[e, :]
            s16 = jnp.where(s16 >= 0, s16, 0.2 * s16)
            w16 = jnp.exp(s16 - cvec)
            rows[e, pl.ds(HC, 16)] = w16
            for j in range(H):
                rows[e, pl.ds(HID * j, HID)] = (
                    rows[e, pl.ds(HID * j, HID)] * w16[j])
            return c2

        lax.fori_loop(0, EB, edge, 0)
        pltpu.sync_copy(rows, acc.at[didx], add=True)
        return carry

    lax.fori_loop(0, NBLK, block, 0)
    plsc.subcore_barrier()

    @pl.when(sid < NS - 1)
    def _():
        pltpu.sync_copy(acc.at[pl.ds(sid * RPT, RPT)],
                        out_hbm.at[cid, pl.ds(sid * RPT, RPT)])

    @pl.when(sid == NS - 1)
    def _():
        rem = N - (NS - 1) * RPT
        pltpu.sync_copy(acc.at[pl.ds((NS - 1) * RPT, rem)],
                        out_hbm.at[cid, pl.ds((NS - 1) * RPT, rem)])


def _sc_layer(tab, dtab, esrc, edst, c16, zeros, H):
    TW = H * HID + 16
    mesh = plsc.VectorSubcoreMesh(core_axis_name="c", subcore_axis_name="s")
    return pl.kernel(
        functools.partial(_sc_body, H),
        out_type=jax.ShapeDtypeStruct((2, N, TW), jnp.float32),
        mesh=mesh,
        compiler_params=pltpu.CompilerParams(use_tc_tiling_on_sc=False),
        scratch_types=[
            pltpu.VMEM((EB, TW), jnp.float32),
            pltpu.VMEM((EB, 16), jnp.float32),
            pltpu.VMEM((EB,), jnp.int32),
            pltpu.VMEM((EB,), jnp.int32),
            pltpu.VMEM((16,), jnp.float32),
            pltpu.VMEM_SHARED((N, TW), jnp.float32),
            pltpu.SemaphoreType.DMA,
            pltpu.SemaphoreType.DMA,
        ],
    )(tab, dtab, esrc, edst, c16, zeros)


def _cvec(cm):
    ch = jnp.maximum(cm[0, :8] + cm[0, 8:], 0.0)
    return jnp.concatenate([ch, jnp.zeros((8,), jnp.float32)])


def kernel(x, edge_index, W1, a_src1, a_dst1, b1, W2, a_src2, a_dst2, b2,
           W3, a_src3, a_dst3, b3, Wc, bc):
    esrc = edge_index[0]
    edst = edge_index[1]
    z144 = jnp.zeros((N - (NS - 1) * RPT, IN + 16), jnp.float32)
    z32 = jnp.zeros((N - (NS - 1) * RPT, HID + 16), jnp.float32)

    tab1, dtab1, cm1 = _tc_pre_first(x, W1, a_src1, a_dst1)
    acc1 = _sc_layer(tab1, dtab1, esrc, edst, _cvec(cm1), z144, 8)

    tab2, dtab2, cm2 = _tc_pre_mid(acc1, b1.reshape(1, IN), W2,
                                   a_src2, a_dst2, 8)
    acc2 = _sc_layer(tab2, dtab2, esrc, edst, _cvec(cm2), z144, 8)

    tab3, dtab3, cm3 = _tc_pre_mid(acc2, b2.reshape(1, IN), W3,
                                   a_src3, a_dst3, 1)
    acc3 = _sc_layer(tab3, dtab3, esrc, edst, _cvec(cm3), z32, 1)

    probs = _tc_fin(acc3, b3.reshape(1, HID), Wc.reshape(1, HID),
                    bc.reshape(1, 1))
    return probs.squeeze(-1)


# speedup vs baseline: 66.5148x; 1.2615x over previous
import functools

import jax
import jax.numpy as jnp
from jax import lax
from jax.experimental import pallas as pl
from jax.experimental.pallas import tpu as pltpu
from jax.experimental.pallas import tpu_sc as plsc

N = 10000
E = 320000
IN = 128
HID = 16

NC = 2
NS = 16
NW = NC * NS
EPT = E // NW
EB = 80
NBLK = EPT // EB
RPT = 624

NB = 400
NG = N // NB


def _elu(x):
    return jnp.where(x > 0, x, jnp.exp(x) - 1.0)


def _pack_tables(h, a_src, a_dst, H, tab_ref, dtab_ref, cm_ref, k):
    nb = h.shape[0]
    h3 = h.reshape(nb, H, HID)
    As = jnp.sum(h3 * a_src[None, :, :], axis=-1)
    Ad = jnp.sum(h3 * a_dst[None, :, :], axis=-1)
    pad = jnp.zeros((nb, 16 - H), dtype=jnp.float32)
    tab_ref[...] = jnp.concatenate([h, As, pad], axis=1)
    dtab_ref[...] = jnp.concatenate([Ad, pad], axis=1)
    if H == 8:
        parts = [jnp.max(As, axis=0), jnp.max(Ad, axis=0)]
    else:
        z8 = jnp.zeros((8 - H,), dtype=jnp.float32)
        parts = [jnp.max(As, axis=0), z8, jnp.max(Ad, axis=0), z8]
    bm = jnp.concatenate(parts).reshape(1, 16)

    @pl.when(k == 0)
    def _():
        cm_ref[...] = bm

    @pl.when(k > 0)
    def _():
        cm_ref[...] = jnp.maximum(cm_ref[...], bm)


def _pre_first_body(x_ref, W_ref, as_ref, ad_ref, tab_ref, dtab_ref, cm_ref):
    k = pl.program_id(0)
    h = jnp.dot(x_ref[...], W_ref[...], preferred_element_type=jnp.float32)
    _pack_tables(h, as_ref[...], ad_ref[...], 8, tab_ref, dtab_ref, cm_ref, k)


def _pre_mid_body(H, acc_ref, b_ref, W_ref, as_ref, ad_ref,
                  tab_ref, dtab_ref, cm_ref):
    k = pl.program_id(0)
    a = acc_ref[0] + acc_ref[1]
    den = a[:, IN:IN + 8]
    den16 = jnp.broadcast_to(den[:, :, None], (NB, 8, 16)).reshape(NB, IN)
    x = _elu(a[:, :IN] / (den16 + 1e-16) + b_ref[...])
    h = jnp.dot(x, W_ref[...], preferred_element_type=jnp.float32)
    _pack_tables(h, as_ref[...], ad_ref[...], H, tab_ref, dtab_ref, cm_ref, k)


def _fin_body(acc_ref, b_ref, wc_ref, bc_ref, out_ref):
    a = acc_ref[0] + acc_ref[1]
    den = jnp.broadcast_to(a[:, HID:HID + 1], (NB, HID))
    x = _elu(a[:, :HID] / (den + 1e-16) + b_ref[...])
    logits = jnp.sum(x * wc_ref[...], axis=1, keepdims=True) + bc_ref[...]
    out_ref[...] = jax.nn.sigmoid(logits)


def _tc_pre_first(x, W, a_src, a_dst):
    return pl.pallas_call(
        _pre_first_body,
        grid=(NG,),
        in_specs=[
            pl.BlockSpec((NB, IN), lambda k: (k, 0)),
            pl.BlockSpec((IN, IN), lambda k: (0, 0)),
            pl.BlockSpec((8, HID), lambda k: (0, 0)),
            pl.BlockSpec((8, HID), lambda k: (0, 0)),
        ],
        out_specs=[
            pl.BlockSpec((NB, IN + 16), lambda k: (k, 0)),
            pl.BlockSpec((NB, 16), lambda k: (k, 0)),
            pl.BlockSpec((1, 16), lambda k: (0, 0)),
        ],
        out_shape=[
            jax.ShapeDtypeStruct((N, IN + 16), jnp.float32),
            jax.ShapeDtypeStruct((N, 16), jnp.float32),
            jax.ShapeDtypeStruct((1, 16), jnp.float32),
        ],
    )(x, W, a_src, a_dst)


def _tc_pre_mid(acc, b_prev, W, a_src, a_dst, H):
    HC = H * HID
    TW = HC + 16
    return pl.pallas_call(
        functools.partial(_pre_mid_body, H),
        grid=(NG,),
        in_specs=[
            pl.BlockSpec((2, NB, IN + 16), lambda k: (0, k, 0)),
            pl.BlockSpec((1, IN), lambda k: (0, 0)),
            pl.BlockSpec((IN, HC), lambda k: (0, 0)),
            pl.BlockSpec((H, HID), lambda k: (0, 0)),
            pl.BlockSpec((H, HID), lambda k: (0, 0)),
        ],
        out_specs=[
            pl.BlockSpec((NB, TW), lambda k: (k, 0)),
            pl.BlockSpec((NB, 16), lambda k: (k, 0)),
            pl.BlockSpec((1, 16), lambda k: (0, 0)),
        ],
        out_shape=[
            jax.ShapeDtypeStruct((N, TW), jnp.float32),
            jax.ShapeDtypeStruct((N, 16), jnp.float32),
            jax.ShapeDtypeStruct((1, 16), jnp.float32),
        ],
    )(acc, b_prev, W, a_src, a_dst)


def _tc_fin(acc, b3, wcT, bc):
    return pl.pallas_call(
        _fin_body,
        grid=(NG,),
        in_specs=[
            pl.BlockSpec((2, NB, HID + 16), lambda k: (0, k, 0)),
            pl.BlockSpec((1, HID), lambda k: (0, 0)),
            pl.BlockSpec((1, HID), lambda k: (0, 0)),
            pl.BlockSpec((1, 1), lambda k: (0, 0)),
        ],
        out_specs=pl.BlockSpec((NB, 1), lambda k: (k, 0)),
        out_shape=jax.ShapeDtypeStruct((N, 1), jnp.float32),
    )(acc, b3, wcT, bc)


def _sc_body(H, tab_hbm, dtab_hbm, esrc_hbm, edst_hbm, c16_hbm, z_hbm,
             out_hbm, rows0, rows1, arows0, arows1, sidx0, sidx1,
             didx0, didx1, c16v, acc, gsem0, gsem1, asem0, asem1):
    HC = H * HID
    cid = lax.axis_index("c")
    sid = lax.axis_index("s")
    wid = sid * NC + cid
    rows_b = (rows0, rows1)
    arows_b = (arows0, arows1)
    sidx_b = (sidx0, sidx1)
    didx_b = (didx0, didx1)
    gsem_b = (gsem0, gsem1)
    asem_b = (asem0, asem1)

    @pl.when(sid < NS - 1)
    def _():
        pltpu.sync_copy(z_hbm.at[pl.ds(0, RPT)], acc.at[pl.ds(sid * RPT, RPT)])

    @pl.when(sid == NS - 1)
    def _():
        pltpu.sync_copy(z_hbm, acc.at[pl.ds((NS - 1) * RPT, N - (NS - 1) * RPT)])

    pltpu.sync_copy(c16_hbm, c16v)
    plsc.subcore_barrier()

    cvec = c16v[...]
    base = wid * EPT

    def stage(b, k):
        off = base + b * EB
        pltpu.sync_copy(esrc_hbm.at[pl.ds(off, EB)], sidx_b[k])
        pltpu.sync_copy(edst_hbm.at[pl.ds(off, EB)], didx_b[k])
        pltpu.async_copy(tab_hbm.at[sidx_b[k]], rows_b[k], gsem_b[k])
        pltpu.async_copy(dtab_hbm.at[didx_b[k]], arows_b[k], asem_b[k])

    def consume(k):
        rows = rows_b[k]
        arows = arows_b[k]
        pltpu.make_async_copy(tab_hbm.at[sidx_b[k]], rows, gsem_b[k]).wait()
        pltpu.make_async_copy(dtab_hbm.at[didx_b[k]], arows, asem_b[k]).wait()

        @pl.loop(0, EB, unroll=2)
        def edge(e):
            s16 = rows[e, pl.ds(HC, 16)] + arows[e, :]
            s16 = jnp.where(s16 >= 0, s16, 0.2 * s16)
            w16 = jnp.exp(s16 - cvec)
            rows[e, pl.ds(HC, 16)] = w16
            for j in range(H):
                rows[e, pl.ds(HID * j, HID)] = (
                    rows[e, pl.ds(HID * j, HID)] * w16[j])

        pltpu.sync_copy(rows, acc.at[didx_b[k]], add=True)

    stage(0, 0)

    def pair(p, carry):
        b = p * 2
        stage(b + 1, 1)
        consume(0)

        @pl.when(b + 2 < NBLK)
        def _():
            stage(b + 2, 0)

        consume(1)
        return carry

    lax.fori_loop(0, NBLK // 2, pair, 0)
    consume(0)
    plsc.subcore_barrier()

    @pl.when(sid < NS - 1)
    def _():
        pltpu.sync_copy(acc.at[pl.ds(sid * RPT, RPT)],
                        out_hbm.at[cid, pl.ds(sid * RPT, RPT)])

    @pl.when(sid == NS - 1)
    def _():
        rem = N - (NS - 1) * RPT
        pltpu.sync_copy(acc.at[pl.ds((NS - 1) * RPT, rem)],
                        out_hbm.at[cid, pl.ds((NS - 1) * RPT, rem)])


def _sc_layer(tab, dtab, esrc, edst, c16, zeros, H):
    TW = H * HID + 16
    mesh = plsc.VectorSubcoreMesh(core_axis_name="c", subcore_axis_name="s")
    return pl.kernel(
        functools.partial(_sc_body, H),
        out_type=jax.ShapeDtypeStruct((2, N, TW), jnp.float32),
        mesh=mesh,
        compiler_params=pltpu.CompilerParams(use_tc_tiling_on_sc=False),
        scratch_types=[
            pltpu.VMEM((EB, TW), jnp.float32),
            pltpu.VMEM((EB, TW), jnp.float32),
            pltpu.VMEM((EB, 16), jnp.float32),
            pltpu.VMEM((EB, 16), jnp.float32),
            pltpu.VMEM((EB,), jnp.int32),
            pltpu.VMEM((EB,), jnp.int32),
            pltpu.VMEM((EB,), jnp.int32),
            pltpu.VMEM((EB,), jnp.int32),
            pltpu.VMEM((16,), jnp.float32),
            pltpu.VMEM_SHARED((N, TW), jnp.float32),
            pltpu.SemaphoreType.DMA,
            pltpu.SemaphoreType.DMA,
            pltpu.SemaphoreType.DMA,
            pltpu.SemaphoreType.DMA,
        ],
    )(tab, dtab, esrc, edst, c16, zeros)


def _cvec(cm):
    ch = jnp.maximum(cm[0, :8] + cm[0, 8:], 0.0)
    return jnp.concatenate([ch, jnp.zeros((8,), jnp.float32)])


def kernel(x, edge_index, W1, a_src1, a_dst1, b1, W2, a_src2, a_dst2, b2,
           W3, a_src3, a_dst3, b3, Wc, bc):
    esrc = edge_index[0]
    edst = edge_index[1]
    z144 = jnp.zeros((N - (NS - 1) * RPT, IN + 16), jnp.float32)
    z32 = jnp.zeros((N - (NS - 1) * RPT, HID + 16), jnp.float32)

    tab1, dtab1, cm1 = _tc_pre_first(x, W1, a_src1, a_dst1)
    acc1 = _sc_layer(tab1, dtab1, esrc, edst, _cvec(cm1), z144, 8)

    tab2, dtab2, cm2 = _tc_pre_mid(acc1, b1.reshape(1, IN), W2,
                                   a_src2, a_dst2, 8)
    acc2 = _sc_layer(tab2, dtab2, esrc, edst, _cvec(cm2), z144, 8)

    tab3, dtab3, cm3 = _tc_pre_mid(acc2, b2.reshape(1, IN), W3,
                                   a_src3, a_dst3, 1)
    acc3 = _sc_layer(tab3, dtab3, esrc, edst, _cvec(cm3), z32, 1)

    probs = _tc_fin(acc3, b3.reshape(1, HID), Wc.reshape(1, HID),
                    bc.reshape(1, 1))
    return probs.squeeze(-1)


# speedup vs baseline: 79.5163x; 1.1955x over previous
import functools

import jax
import jax.numpy as jnp
from jax import lax
from jax.experimental import pallas as pl
from jax.experimental.pallas import tpu as pltpu
from jax.experimental.pallas import tpu_sc as plsc

N = 10000
E = 320000
IN = 128
HID = 16

NC = 2
NS = 16
NW = NC * NS
EPT = E // NW
EB = 100
NBLK = EPT // EB
CH = 10
NCH = NBLK // CH
RPT = 624

NB = 400
NG = N // NB


def _elu(x):
    return jnp.where(x > 0, x, jnp.exp(x) - 1.0)


def _pack_tables(h, a_src, a_dst, H, tab_ref, dtab_ref, cm_ref, k):
    nb = h.shape[0]
    h3 = h.reshape(nb, H, HID)
    As = jnp.sum(h3 * a_src[None, :, :], axis=-1)
    Ad = jnp.sum(h3 * a_dst[None, :, :], axis=-1)
    pad = jnp.zeros((nb, 16 - H), dtype=jnp.float32)
    tab_ref[...] = jnp.concatenate([h, As, pad], axis=1)
    dtab_ref[...] = jnp.concatenate([Ad, pad], axis=1)
    if H == 8:
        parts = [jnp.max(As, axis=0), jnp.max(Ad, axis=0)]
    else:
        z8 = jnp.zeros((8 - H,), dtype=jnp.float32)
        parts = [jnp.max(As, axis=0), z8, jnp.max(Ad, axis=0), z8]
    bm = jnp.concatenate(parts).reshape(1, 16)

    @pl.when(k == 0)
    def _():
        cm_ref[...] = bm

    @pl.when(k > 0)
    def _():
        cm_ref[...] = jnp.maximum(cm_ref[...], bm)


def _pre_first_body(x_ref, W_ref, as_ref, ad_ref, tab_ref, dtab_ref, cm_ref):
    k = pl.program_id(0)
    h = jnp.dot(x_ref[...], W_ref[...], preferred_element_type=jnp.float32)
    _pack_tables(h, as_ref[...], ad_ref[...], 8, tab_ref, dtab_ref, cm_ref, k)


def _pre_mid_body(H, acc_ref, b_ref, W_ref, as_ref, ad_ref,
                  tab_ref, dtab_ref, cm_ref):
    k = pl.program_id(0)
    a = acc_ref[0] + acc_ref[1]
    den = a[:, IN:IN + 8]
    den16 = jnp.broadcast_to(den[:, :, None], (NB, 8, 16)).reshape(NB, IN)
    x = _elu(a[:, :IN] / (den16 + 1e-16) + b_ref[...])
    h = jnp.dot(x, W_ref[...], preferred_element_type=jnp.float32)
    _pack_tables(h, as_ref[...], ad_ref[...], H, tab_ref, dtab_ref, cm_ref, k)


def _fin_body(acc_ref, b_ref, wc_ref, bc_ref, out_ref):
    a = acc_ref[0] + acc_ref[1]
    den = jnp.broadcast_to(a[:, HID:HID + 1], (NB, HID))
    x = _elu(a[:, :HID] / (den + 1e-16) + b_ref[...])
    logits = jnp.sum(x * wc_ref[...], axis=1, keepdims=True) + bc_ref[...]
    out_ref[...] = jax.nn.sigmoid(logits)


def _tc_pre_first(x, W, a_src, a_dst):
    return pl.pallas_call(
        _pre_first_body,
        grid=(NG,),
        in_specs=[
            pl.BlockSpec((NB, IN), lambda k: (k, 0)),
            pl.BlockSpec((IN, IN), lambda k: (0, 0)),
            pl.BlockSpec((8, HID), lambda k: (0, 0)),
            pl.BlockSpec((8, HID), lambda k: (0, 0)),
        ],
        out_specs=[
            pl.BlockSpec((NB, IN + 16), lambda k: (k, 0)),
            pl.BlockSpec((NB, 16), lambda k: (k, 0)),
            pl.BlockSpec((1, 16), lambda k: (0, 0)),
        ],
        out_shape=[
            jax.ShapeDtypeStruct((N, IN + 16), jnp.float32),
            jax.ShapeDtypeStruct((N, 16), jnp.float32),
            jax.ShapeDtypeStruct((1, 16), jnp.float32),
        ],
    )(x, W, a_src, a_dst)


def _tc_pre_mid(acc, b_prev, W, a_src, a_dst, H):
    HC = H * HID
    TW = HC + 16
    return pl.pallas_call(
        functools.partial(_pre_mid_body, H),
        grid=(NG,),
        in_specs=[
            pl.BlockSpec((2, NB, IN + 16), lambda k: (0, k, 0)),
            pl.BlockSpec((1, IN), lambda k: (0, 0)),
            pl.BlockSpec((IN, HC), lambda k: (0, 0)),
            pl.BlockSpec((H, HID), lambda k: (0, 0)),
            pl.BlockSpec((H, HID), lambda k: (0, 0)),
        ],
        out_specs=[
            pl.BlockSpec((NB, TW), lambda k: (k, 0)),
            pl.BlockSpec((NB, 16), lambda k: (k, 0)),
            pl.BlockSpec((1, 16), lambda k: (0, 0)),
        ],
        out_shape=[
            jax.ShapeDtypeStruct((N, TW), jnp.float32),
            jax.ShapeDtypeStruct((N, 16), jnp.float32),
            jax.ShapeDtypeStruct((1, 16), jnp.float32),
        ],
    )(acc, b_prev, W, a_src, a_dst)


def _tc_fin(acc, b3, wcT, bc):
    return pl.pallas_call(
        _fin_body,
        grid=(NG,),
        in_specs=[
            pl.BlockSpec((2, NB, HID + 16), lambda k: (0, k, 0)),
            pl.BlockSpec((1, HID), lambda k: (0, 0)),
            pl.BlockSpec((1, HID), lambda k: (0, 0)),
            pl.BlockSpec((1, 1), lambda k: (0, 0)),
        ],
        out_specs=pl.BlockSpec((NB, 1), lambda k: (k, 0)),
        out_shape=jax.ShapeDtypeStruct((N, 1), jnp.float32),
    )(acc, b3, wcT, bc)


def _sc_body(H, tab_hbm, dtab_hbm, esrc_hbm, edst_hbm, c16_hbm, z_hbm,
             out_hbm, rows0, rows1, arows0, arows1, sidx2, didx2,
             c16v, acc, gsem0, gsem1, asem0, asem1):
    HC = H * HID
    cid = lax.axis_index("c")
    sid = lax.axis_index("s")
    wid = sid * NC + cid
    rows_b = (rows0, rows1)
    arows_b = (arows0, arows1)
    gsem_b = (gsem0, gsem1)
    asem_b = (asem0, asem1)

    @pl.when(sid < NS - 1)
    def _():
        pltpu.sync_copy(z_hbm.at[pl.ds(0, RPT)], acc.at[pl.ds(sid * RPT, RPT)])

    @pl.when(sid == NS - 1)
    def _():
        pltpu.sync_copy(z_hbm, acc.at[pl.ds((NS - 1) * RPT, N - (NS - 1) * RPT)])

    pltpu.sync_copy(c16_hbm, c16v)
    plsc.subcore_barrier()

    cvec = c16v[...]

    def stage(b, k):
        pltpu.async_copy(tab_hbm.at[sidx2.at[b]], rows_b[k], gsem_b[k])
        pltpu.async_copy(dtab_hbm.at[didx2.at[b]], arows_b[k], asem_b[k])

    def consume(b, k):
        rows = rows_b[k]
        arows = arows_b[k]
        pltpu.make_async_copy(tab_hbm.at[sidx2.at[b]], rows,
                              gsem_b[k]).wait()
        pltpu.make_async_copy(dtab_hbm.at[didx2.at[b]], arows,
                              asem_b[k]).wait()

        @pl.loop(0, EB, unroll=4)
        def edge(e):
            s16 = rows[e, pl.ds(HC, 16)] + arows[e, :]
            s16 = jnp.where(s16 >= 0, s16, 0.2 * s16)
            w16 = jnp.exp(s16 - cvec)
            rows[e, pl.ds(HC, 16)] = w16
            for j in range(H):
                rows[e, pl.ds(HID * j, HID)] = (
                    rows[e, pl.ds(HID * j, HID)] * w16[j])

        pltpu.sync_copy(rows, acc.at[didx2.at[b]], add=True)

    def chunk(q, carry):
        pltpu.sync_copy(esrc_hbm.at[wid, q], sidx2)
        pltpu.sync_copy(edst_hbm.at[wid, q], didx2)
        stage(0, 0)

        def pair(p, c2):
            b = p * 2
            stage(b + 1, 1)
            consume(b, 0)

            @pl.when(b + 2 < CH)
            def _():
                stage(b + 2, 0)

            consume(b + 1, 1)
            return c2

        lax.fori_loop(0, CH // 2, pair, 0)
        return carry

    lax.fori_loop(0, NCH, chunk, 0)
    plsc.subcore_barrier()

    @pl.when(sid < NS - 1)
    def _():
        pltpu.sync_copy(acc.at[pl.ds(sid * RPT, RPT)],
                        out_hbm.at[cid, pl.ds(sid * RPT, RPT)])

    @pl.when(sid == NS - 1)
    def _():
        rem = N - (NS - 1) * RPT
        pltpu.sync_copy(acc.at[pl.ds((NS - 1) * RPT, rem)],
                        out_hbm.at[cid, pl.ds((NS - 1) * RPT, rem)])


def _sc_layer(tab, dtab, esrc, edst, c16, zeros, H):
    TW = H * HID + 16
    mesh = plsc.VectorSubcoreMesh(core_axis_name="c", subcore_axis_name="s")
    return pl.kernel(
        functools.partial(_sc_body, H),
        out_type=jax.ShapeDtypeStruct((2, N, TW), jnp.float32),
        mesh=mesh,
        compiler_params=pltpu.CompilerParams(use_tc_tiling_on_sc=False),
        scratch_types=[
            pltpu.VMEM((EB, TW), jnp.float32),
            pltpu.VMEM((EB, TW), jnp.float32),
            pltpu.VMEM((EB, 16), jnp.float32),
            pltpu.VMEM((EB, 16), jnp.float32),
            pltpu.VMEM((CH, EB), jnp.int32),
            pltpu.VMEM((CH, EB), jnp.int32),
            pltpu.VMEM((16,), jnp.float32),
            pltpu.VMEM_SHARED((N, TW), jnp.float32),
            pltpu.SemaphoreType.DMA,
            pltpu.SemaphoreType.DMA,
            pltpu.SemaphoreType.DMA,
            pltpu.SemaphoreType.DMA,
        ],
    )(tab, dtab, esrc, edst, c16, zeros)


def _cvec(cm):
    ch = jnp.maximum(cm[0, :8] + cm[0, 8:], 0.0)
    return jnp.concatenate([ch, jnp.zeros((8,), jnp.float32)])


def kernel(x, edge_index, W1, a_src1, a_dst1, b1, W2, a_src2, a_dst2, b2,
           W3, a_src3, a_dst3, b3, Wc, bc):
    esrc = edge_index[0].reshape(NW, NCH, CH, EB)
    edst = edge_index[1].reshape(NW, NCH, CH, EB)
    z144 = jnp.zeros((N - (NS - 1) * RPT, IN + 16), jnp.float32)
    z32 = jnp.zeros((N - (NS - 1) * RPT, HID + 16), jnp.float32)

    tab1, dtab1, cm1 = _tc_pre_first(x, W1, a_src1, a_dst1)
    acc1 = _sc_layer(tab1, dtab1, esrc, edst, _cvec(cm1), z144, 8)

    tab2, dtab2, cm2 = _tc_pre_mid(acc1, b1.reshape(1, IN), W2,
                                   a_src2, a_dst2, 8)
    acc2 = _sc_layer(tab2, dtab2, esrc, edst, _cvec(cm2), z144, 8)

    tab3, dtab3, cm3 = _tc_pre_mid(acc2, b2.reshape(1, IN), W3,
                                   a_src3, a_dst3, 1)
    acc3 = _sc_layer(tab3, dtab3, esrc, edst, _cvec(cm3), z32, 1)

    probs = _tc_fin(acc3, b3.reshape(1, HID), Wc.reshape(1, HID),
                    bc.reshape(1, 1))
    return probs.squeeze(-1)


# speedup vs baseline: 80.4466x; 1.0117x over previous
import functools

import jax
import jax.numpy as jnp
from jax import lax
from jax.experimental import pallas as pl
from jax.experimental.pallas import tpu as pltpu
from jax.experimental.pallas import tpu_sc as plsc

N = 10000
E = 320000
IN = 128
HID = 16

NC = 2
NS = 16
NW = NC * NS
EPT = E // NW
EB = 100
NBLK = EPT // EB
CH = 20
NCH = NBLK // CH
RPT = 624

NB = 400
NG = N // NB


def _elu(x):
    return jnp.where(x > 0, x, jnp.exp(x) - 1.0)


def _pack_tables(h, a_src, a_dst, H, tab_ref, dtab_ref, cm_ref, k):
    nb = h.shape[0]
    h3 = h.reshape(nb, H, HID)
    As = jnp.sum(h3 * a_src[None, :, :], axis=-1)
    Ad = jnp.sum(h3 * a_dst[None, :, :], axis=-1)
    pad = jnp.zeros((nb, 16 - H), dtype=jnp.float32)
    tab_ref[...] = jnp.concatenate([h, As, pad], axis=1)
    dtab_ref[...] = jnp.concatenate([Ad, pad], axis=1)
    if H == 8:
        parts = [jnp.max(As, axis=0), jnp.max(Ad, axis=0)]
    else:
        z8 = jnp.zeros((8 - H,), dtype=jnp.float32)
        parts = [jnp.max(As, axis=0), z8, jnp.max(Ad, axis=0), z8]
    bm = jnp.concatenate(parts).reshape(1, 16)

    @pl.when(k == 0)
    def _():
        cm_ref[...] = bm

    @pl.when(k > 0)
    def _():
        cm_ref[...] = jnp.maximum(cm_ref[...], bm)


def _pre_first_body(x_ref, W_ref, as_ref, ad_ref, tab_ref, dtab_ref, cm_ref):
    k = pl.program_id(0)
    h = jnp.dot(x_ref[...], W_ref[...], preferred_element_type=jnp.float32)
    _pack_tables(h, as_ref[...], ad_ref[...], 8, tab_ref, dtab_ref, cm_ref, k)


def _pre_mid_body(H, acc_ref, b_ref, W_ref, as_ref, ad_ref,
                  tab_ref, dtab_ref, cm_ref):
    k = pl.program_id(0)
    a = acc_ref[0] + acc_ref[1]
    den = a[:, IN:IN + 8]
    den16 = jnp.broadcast_to(den[:, :, None], (NB, 8, 16)).reshape(NB, IN)
    x = _elu(a[:, :IN] / (den16 + 1e-16) + b_ref[...])
    h = jnp.dot(x, W_ref[...], preferred_element_type=jnp.float32)
    _pack_tables(h, as_ref[...], ad_ref[...], H, tab_ref, dtab_ref, cm_ref, k)


def _fin_body(acc_ref, b_ref, wc_ref, bc_ref, out_ref):
    a = acc_ref[0] + acc_ref[1]
    den = jnp.broadcast_to(a[:, HID:HID + 1], (NB, HID))
    x = _elu(a[:, :HID] / (den + 1e-16) + b_ref[...])
    logits = jnp.sum(x * wc_ref[...], axis=1, keepdims=True) + bc_ref[...]
    out_ref[...] = jax.nn.sigmoid(logits)


def _tc_pre_first(x, W, a_src, a_dst):
    return pl.pallas_call(
        _pre_first_body,
        grid=(NG,),
        in_specs=[
            pl.BlockSpec((NB, IN), lambda k: (k, 0)),
            pl.BlockSpec((IN, IN), lambda k: (0, 0)),
            pl.BlockSpec((8, HID), lambda k: (0, 0)),
            pl.BlockSpec((8, HID), lambda k: (0, 0)),
        ],
        out_specs=[
            pl.BlockSpec((NB, IN + 16), lambda k: (k, 0)),
            pl.BlockSpec((NB, 16), lambda k: (k, 0)),
            pl.BlockSpec((1, 16), lambda k: (0, 0)),
        ],
        out_shape=[
            jax.ShapeDtypeStruct((N, IN + 16), jnp.float32),
            jax.ShapeDtypeStruct((N, 16), jnp.float32),
            jax.ShapeDtypeStruct((1, 16), jnp.float32),
        ],
    )(x, W, a_src, a_dst)


def _tc_pre_mid(acc, b_prev, W, a_src, a_dst, H):
    HC = H * HID
    TW = HC + 16
    return pl.pallas_call(
        functools.partial(_pre_mid_body, H),
        grid=(NG,),
        in_specs=[
            pl.BlockSpec((2, NB, IN + 16), lambda k: (0, k, 0)),
            pl.BlockSpec((1, IN), lambda k: (0, 0)),
            pl.BlockSpec((IN, HC), lambda k: (0, 0)),
            pl.BlockSpec((H, HID), lambda k: (0, 0)),
            pl.BlockSpec((H, HID), lambda k: (0, 0)),
        ],
        out_specs=[
            pl.BlockSpec((NB, TW), lambda k: (k, 0)),
            pl.BlockSpec((NB, 16), lambda k: (k, 0)),
            pl.BlockSpec((1, 16), lambda k: (0, 0)),
        ],
        out_shape=[
            jax.ShapeDtypeStruct((N, TW), jnp.float32),
            jax.ShapeDtypeStruct((N, 16), jnp.float32),
            jax.ShapeDtypeStruct((1, 16), jnp.float32),
        ],
    )(acc, b_prev, W, a_src, a_dst)


def _tc_fin(acc, b3, wcT, bc):
    return pl.pallas_call(
        _fin_body,
        grid=(NG,),
        in_specs=[
            pl.BlockSpec((2, NB, HID + 16), lambda k: (0, k, 0)),
            pl.BlockSpec((1, HID), lambda k: (0, 0)),
            pl.BlockSpec((1, HID), lambda k: (0, 0)),
            pl.BlockSpec((1, 1), lambda k: (0, 0)),
        ],
        out_specs=pl.BlockSpec((NB, 1), lambda k: (k, 0)),
        out_shape=jax.ShapeDtypeStruct((N, 1), jnp.float32),
    )(acc, b3, wcT, bc)


def _sc_body(H, tab_hbm, dtab_hbm, esrc_hbm, edst_hbm, c16_hbm, z_hbm,
             out_hbm, rows0, rows1, arows0, arows1, sidx2, didx2,
             c16v, acc, gsem0, gsem1, asem0, asem1, ssem0, ssem1):
    HC = H * HID
    cid = lax.axis_index("c")
    sid = lax.axis_index("s")
    wid = sid * NC + cid
    rows_b = (rows0, rows1)
    arows_b = (arows0, arows1)
    gsem_b = (gsem0, gsem1)
    asem_b = (asem0, asem1)
    ssem_b = (ssem0, ssem1)

    @pl.when(sid < NS - 1)
    def _():
        pltpu.sync_copy(z_hbm.at[pl.ds(0, RPT)], acc.at[pl.ds(sid * RPT, RPT)])

    @pl.when(sid == NS - 1)
    def _():
        pltpu.sync_copy(z_hbm, acc.at[pl.ds((NS - 1) * RPT, N - (NS - 1) * RPT)])

    pltpu.sync_copy(c16_hbm, c16v)
    plsc.subcore_barrier()

    cvec = c16v[...]

    def stage(b, k):
        pltpu.async_copy(tab_hbm.at[sidx2.at[b]], rows_b[k], gsem_b[k])
        pltpu.async_copy(dtab_hbm.at[didx2.at[b]], arows_b[k], asem_b[k])

    def consume(b, k):
        rows = rows_b[k]
        arows = arows_b[k]
        pltpu.make_async_copy(tab_hbm.at[sidx2.at[b]], rows,
                              gsem_b[k]).wait()
        pltpu.make_async_copy(dtab_hbm.at[didx2.at[b]], arows,
                              asem_b[k]).wait()

        @pl.loop(0, EB, unroll=4)
        def edge(e):
            s16 = rows[e, pl.ds(HC, 16)] + arows[e, :]
            s16 = jnp.where(s16 >= 0, s16, 0.2 * s16)
            w16 = jnp.exp(s16 - cvec)
            rows[e, pl.ds(HC, 16)] = w16
            for j in range(H):
                rows[e, pl.ds(HID * j, HID)] = (
                    rows[e, pl.ds(HID * j, HID)] * w16[j])

        pltpu.async_copy(rows, acc.at[didx2.at[b]], ssem_b[k], add=True)

    def wait_scat(b, k):
        pltpu.make_async_copy(rows_b[k], acc.at[didx2.at[b]],
                              ssem_b[k]).wait()

    def chunk(q, carry):
        pltpu.sync_copy(esrc_hbm.at[wid, q], sidx2)
        pltpu.sync_copy(edst_hbm.at[wid, q], didx2)
        stage(0, 0)
        stage(1, 1)

        def pair(p, c2):
            b = p * 2
            consume(b, 0)
            consume(b + 1, 1)
            wait_scat(b, 0)

            @pl.when(b + 2 < CH)
            def _():
                stage(b + 2, 0)

            wait_scat(b + 1, 1)

            @pl.when(b + 3 < CH)
            def _():
                stage(b + 3, 1)

            return c2

        lax.fori_loop(0, CH // 2, pair, 0)
        return carry

    lax.fori_loop(0, NCH, chunk, 0)
    plsc.subcore_barrier()

    @pl.when(sid < NS - 1)
    def _():
        pltpu.sync_copy(acc.at[pl.ds(sid * RPT, RPT)],
                        out_hbm.at[cid, pl.ds(sid * RPT, RPT)])

    @pl.when(sid == NS - 1)
    def _():
        rem = N - (NS - 1) * RPT
        pltpu.sync_copy(acc.at[pl.ds((NS - 1) * RPT, rem)],
                        out_hbm.at[cid, pl.ds((NS - 1) * RPT, rem)])


def _sc_layer(tab, dtab, esrc, edst, c16, zeros, H):
    TW = H * HID + 16
    mesh = plsc.VectorSubcoreMesh(core_axis_name="c", subcore_axis_name="s")
    return pl.kernel(
        functools.partial(_sc_body, H),
        out_type=jax.ShapeDtypeStruct((2, N, TW), jnp.float32),
        mesh=mesh,
        compiler_params=pltpu.CompilerParams(use_tc_tiling_on_sc=False),
        scratch_types=[
            pltpu.VMEM((EB, TW), jnp.float32),
            pltpu.VMEM((EB, TW), jnp.float32),
            pltpu.VMEM((EB, 16), jnp.float32),
            pltpu.VMEM((EB, 16), jnp.float32),
            pltpu.VMEM((CH, EB), jnp.int32),
            pltpu.VMEM((CH, EB), jnp.int32),
            pltpu.VMEM((16,), jnp.float32),
            pltpu.VMEM_SHARED((N, TW), jnp.float32),
            pltpu.SemaphoreType.DMA,
            pltpu.SemaphoreType.DMA,
            pltpu.SemaphoreType.DMA,
            pltpu.SemaphoreType.DMA,
            pltpu.SemaphoreType.DMA,
            pltpu.SemaphoreType.DMA,
        ],
    )(tab, dtab, esrc, edst, c16, zeros)


def _cvec(cm):
    ch = jnp.maximum(cm[0, :8] + cm[0, 8:], 0.0)
    return jnp.concatenate([ch, jnp.zeros((8,), jnp.float32)])


def kernel(x, edge_index, W1, a_src1, a_dst1, b1, W2, a_src2, a_dst2, b2,
           W3, a_src3, a_dst3, b3, Wc, bc):
    esrc = edge_index[0].reshape(NW, NCH, CH, EB)
    edst = edge_index[1].reshape(NW, NCH, CH, EB)
    z144 = jnp.zeros((N - (NS - 1) * RPT, IN + 16), jnp.float32)
    z32 = jnp.zeros((N - (NS - 1) * RPT, HID + 16), jnp.float32)

    tab1, dtab1, cm1 = _tc_pre_first(x, W1, a_src1, a_dst1)
    acc1 = _sc_layer(tab1, dtab1, esrc, edst, _cvec(cm1), z144, 8)

    tab2, dtab2, cm2 = _tc_pre_mid(acc1, b1.reshape(1, IN), W2,
                                   a_src2, a_dst2, 8)
    acc2 = _sc_layer(tab2, dtab2, esrc, edst, _cvec(cm2), z144, 8)

    tab3, dtab3, cm3 = _tc_pre_mid(acc2, b2.reshape(1, IN), W3,
                                   a_src3, a_dst3, 1)
    acc3 = _sc_layer(tab3, dtab3, esrc, edst, _cvec(cm3), z32, 1)

    probs = _tc_fin(acc3, b3.reshape(1, HID), Wc.reshape(1, HID),
                    bc.reshape(1, 1))
    return probs.squeeze(-1)
